# baseline (speedup 1.0000x reference)
import functools

import jax
import jax.numpy as jnp
from jax import lax
from jax.experimental import pallas as pl
from jax.experimental.pallas import tpu as pltpu
from jax.experimental.pallas import tpu_sc as plsc
from jax.sharding import PartitionSpec as P

_L = 4096
_M = 1024
_W = 128


def _fps_kernel(n, cx_ref, cy_ref, cz_ref,
                gidx_ref, ox_ref, oy_ref, oz_ref,
                dist_ref, lxr, lyr, lzr):
    j = pl.program_id(0)
    iota = lax.broadcasted_iota(jnp.int32, (n, _L), 1)
    lane = lax.broadcasted_iota(jnp.int32, (n, _W), 1)
    base = lax.broadcasted_iota(jnp.int32, (n, 1), 0) * _L
    is0 = j == 0

    @pl.when(is0)
    def _init():
        dist_ref[...] = jnp.full((n, _L), jnp.inf, jnp.float32)
        lxr[...] = cx_ref[:, 0:1]
        lyr[...] = cy_ref[:, 0:1]
        lzr[...] = cz_ref[:, 0:1]

    lx0 = lxr[...]
    ly0 = lyr[...]
    lz0 = lzr[...]
    seed = is0 & (lane == 0)
    gbuf0 = jnp.where(seed, base, 0)
    xbuf0 = jnp.where(seed, lx0, 0.0)
    ybuf0 = jnp.where(seed, ly0, 0.0)
    zbuf0 = jnp.where(seed, lz0, 0.0)

    def body(k, carry):
        gbuf, xbuf, ybuf, zbuf, lx, ly, lz = carry
        cx = cx_ref[...]
        cy = cy_ref[...]
        cz = cz_ref[...]
        dx = cx - lx
        dy = cy - ly
        dz = cz - lz
        d = dx * dx + dy * dy + dz * dz
        dm = jnp.minimum(dist_ref[...], d)
        dist_ref[...] = dm
        mx = jnp.max(dm, axis=1, keepdims=True)
        nxt = jnp.min(jnp.where(dm == mx, iota, _L), axis=1, keepdims=True)
        oh = iota == nxt
        nlx = jnp.sum(jnp.where(oh, cx, 0.0), axis=1, keepdims=True)
        nly = jnp.sum(jnp.where(oh, cy, 0.0), axis=1, keepdims=True)
        nlz = jnp.sum(jnp.where(oh, cz, 0.0), axis=1, keepdims=True)
        m = lane == k
        return (jnp.where(m, nxt + base, gbuf),
                jnp.where(m, nlx, xbuf),
                jnp.where(m, nly, ybuf),
                jnp.where(m, nlz, zbuf),
                nlx, nly, nlz)

    start = jnp.where(is0, 1, 0)
    gbuf, xbuf, ybuf, zbuf, lx, ly, lz = lax.fori_loop(
        start, _W, body, (gbuf0, xbuf0, ybuf0, zbuf0, lx0, ly0, lz0))

    gidx_ref[...] = gbuf
    ox_ref[...] = xbuf
    oy_ref[...] = ybuf
    oz_ref[...] = zbuf
    lxr[...] = lx
    lyr[...] = ly
    lzr[...] = lz


def _fps(cx, cy, cz):
    n = cx.shape[0]
    out_block = pl.BlockSpec((n, _W), lambda j: (0, j))
    return pl.pallas_call(
        functools.partial(_fps_kernel, n),
        grid=(_M // _W,),
        in_specs=[pl.BlockSpec((n, _L), lambda j: (0, 0))] * 3,
        out_specs=[out_block] * 4,
        out_shape=[
            jax.ShapeDtypeStruct((n, _M), jnp.int32),
            jax.ShapeDtypeStruct((n, _M), jnp.float32),
            jax.ShapeDtypeStruct((n, _M), jnp.float32),
            jax.ShapeDtypeStruct((n, _M), jnp.float32),
        ],
        scratch_shapes=[pltpu.VMEM((n, _L), jnp.float32),
                        pltpu.VMEM((n, 1), jnp.float32),
                        pltpu.VMEM((n, 1), jnp.float32),
                        pltpu.VMEM((n, 1), jnp.float32)],
    )(cx, cy, cz)


def _sc_gather(table, idx):
    num_idx = idx.shape[1]
    depth = table.shape[1]
    window = 128
    mesh = plsc.VectorSubcoreMesh(core_axis_name="core",
                                  subcore_axis_name="subcore")

    @pl.kernel(out_type=jax.ShapeDtypeStruct((num_idx, depth), table.dtype),
               mesh=mesh)
    def gather_kernel(x_hbm, i_hbm, o_hbm):
        def body(i_vmem, o_vmem):
            pltpu.sync_copy(x_hbm.at[i_vmem.at[0]], o_vmem)

        pltpu.emit_pipeline(
            body,
            grid=(num_idx // window,),
            in_specs=[pl.BlockSpec((1, window), index_map=lambda i: (0, i))],
            out_specs=[pl.BlockSpec((window, depth),
                                    index_map=lambda i: (i, 0))],
            core_axis_name=("core", "subcore"),
            dimension_semantics=(pltpu.PARALLEL,),
        )(i_hbm, o_hbm)

    return gather_kernel(table, idx)


def _sample_shard(coords, features):
    n = coords.shape[0]
    cx = coords[:, :, 0]
    cy = coords[:, :, 1]
    cz = coords[:, :, 2]
    gidx, ox, oy, oz = _fps(cx, cy, cz)
    coords_out = jnp.stack([ox, oy, oz], axis=-1)
    depth = features.shape[-1]
    feats_flat = features.reshape(n * _L, depth)
    feats_pad = jnp.pad(feats_flat, ((0, 0), (0, 128 - depth)))
    feats_out = _sc_gather(feats_pad, gidx.reshape(1, n * _M))
    feats_out = feats_out[:, :depth].reshape(n, _M, depth)
    return coords_out, feats_out


def kernel(coords, features):
    num_dev = len(jax.devices())
    if num_dev >= 2 and coords.shape[0] % 2 == 0:
        mesh = jax.make_mesh((2,), ("b",))
        sharding = jax.NamedSharding(mesh, P("b"))
        coords = jax.reshard(coords, sharding)
        features = jax.reshard(features, sharding)
        fn = jax.shard_map(_sample_shard, mesh=mesh,
                           in_specs=(P("b"), P("b")),
                           out_specs=(P("b"), P("b")),
                           check_vma=False)
        return fn(coords, features)
    return _sample_shard(coords, features)


# speedup vs baseline: 1.5998x; 1.5998x over previous
import functools

import jax
import jax.numpy as jnp
from jax import lax
from jax.experimental import pallas as pl
from jax.experimental.pallas import tpu as pltpu
from jax.experimental.pallas import tpu_sc as plsc
from jax.sharding import PartitionSpec as P

_L = 4096
_M = 1024
_W = 128


_H = 8


def _fps_kernel(n, cx_ref, cy_ref, cz_ref,
                gidx_ref, ox_ref, oy_ref, oz_ref,
                dist_ref, lxr, lyr, lzr):
    j = pl.program_id(0)
    halves = [(h, min(_H, n - h)) for h in range(0, n, _H)]
    is0 = j == 0

    @pl.when(is0)
    def _init():
        dist_ref[...] = jnp.full((n, _L), jnp.inf, jnp.float32)
        lxr[...] = cx_ref[:, 0:1]
        lyr[...] = cy_ref[:, 0:1]
        lzr[...] = cz_ref[:, 0:1]

    def half_state0(h, hn):
        rows = slice(h, h + hn)
        lane = lax.broadcasted_iota(jnp.int32, (hn, _W), 1)
        base = (lax.broadcasted_iota(jnp.int32, (hn, 1), 0) + h) * _L
        lx0 = lxr[rows, :]
        ly0 = lyr[rows, :]
        lz0 = lzr[rows, :]
        seed = is0 & (lane == 0)
        return (jnp.where(seed, base, 0),
                jnp.where(seed, lx0, 0.0),
                jnp.where(seed, ly0, 0.0),
                jnp.where(seed, lz0, 0.0),
                lx0, ly0, lz0)

    def half_step(k, h, hn, st):
        gbuf, xbuf, ybuf, zbuf, lx, ly, lz = st
        rows = slice(h, h + hn)
        iota = lax.broadcasted_iota(jnp.int32, (hn, _L), 1)
        lane = lax.broadcasted_iota(jnp.int32, (hn, _W), 1)
        base = (lax.broadcasted_iota(jnp.int32, (hn, 1), 0) + h) * _L
        cx = cx_ref[rows, :]
        cy = cy_ref[rows, :]
        cz = cz_ref[rows, :]
        dx = cx - lx
        dy = cy - ly
        dz = cz - lz
        d = dx * dx + dy * dy + dz * dz
        dm = jnp.minimum(dist_ref[rows, :], d)
        dist_ref[rows, :] = dm
        mx = jnp.max(dm, axis=1, keepdims=True)
        nxt = jnp.min(jnp.where(dm == mx, iota, _L), axis=1, keepdims=True)
        oh = iota == nxt
        nlx = jnp.sum(jnp.where(oh, cx, 0.0), axis=1, keepdims=True)
        nly = jnp.sum(jnp.where(oh, cy, 0.0), axis=1, keepdims=True)
        nlz = jnp.sum(jnp.where(oh, cz, 0.0), axis=1, keepdims=True)
        m = lane == k
        return (jnp.where(m, nxt + base, gbuf),
                jnp.where(m, nlx, xbuf),
                jnp.where(m, nly, ybuf),
                jnp.where(m, nlz, zbuf),
                nlx, nly, nlz)

    def body(k, carry):
        return tuple(half_step(k, h, hn, st)
                     for (h, hn), st in zip(halves, carry))

    start = jnp.where(is0, 1, 0)
    final = lax.fori_loop(
        start, _W, body,
        tuple(half_state0(h, hn) for h, hn in halves))

    for (h, hn), st in zip(halves, final):
        gbuf, xbuf, ybuf, zbuf, lx, ly, lz = st
        rows = slice(h, h + hn)
        gidx_ref[rows, :] = gbuf
        ox_ref[rows, :] = xbuf
        oy_ref[rows, :] = ybuf
        oz_ref[rows, :] = zbuf
        lxr[rows, :] = lx
        lyr[rows, :] = ly
        lzr[rows, :] = lz


def _fps(cx, cy, cz):
    n = cx.shape[0]
    out_block = pl.BlockSpec((n, _W), lambda j: (0, j))
    return pl.pallas_call(
        functools.partial(_fps_kernel, n),
        grid=(_M // _W,),
        in_specs=[pl.BlockSpec((n, _L), lambda j: (0, 0))] * 3,
        out_specs=[out_block] * 4,
        out_shape=[
            jax.ShapeDtypeStruct((n, _M), jnp.int32),
            jax.ShapeDtypeStruct((n, _M), jnp.float32),
            jax.ShapeDtypeStruct((n, _M), jnp.float32),
            jax.ShapeDtypeStruct((n, _M), jnp.float32),
        ],
        scratch_shapes=[pltpu.VMEM((n, _L), jnp.float32),
                        pltpu.VMEM((n, 1), jnp.float32),
                        pltpu.VMEM((n, 1), jnp.float32),
                        pltpu.VMEM((n, 1), jnp.float32)],
    )(cx, cy, cz)


def _sc_gather(table, idx):
    num_idx = idx.shape[1]
    depth = table.shape[1]
    window = 128
    mesh = plsc.VectorSubcoreMesh(core_axis_name="core",
                                  subcore_axis_name="subcore")

    @pl.kernel(out_type=jax.ShapeDtypeStruct((num_idx, depth), table.dtype),
               mesh=mesh)
    def gather_kernel(x_hbm, i_hbm, o_hbm):
        def body(i_vmem, o_vmem):
            pltpu.sync_copy(x_hbm.at[i_vmem.at[0]], o_vmem)

        pltpu.emit_pipeline(
            body,
            grid=(num_idx // window,),
            in_specs=[pl.BlockSpec((1, window), index_map=lambda i: (0, i))],
            out_specs=[pl.BlockSpec((window, depth),
                                    index_map=lambda i: (i, 0))],
            core_axis_name=("core", "subcore"),
            dimension_semantics=(pltpu.PARALLEL,),
        )(i_hbm, o_hbm)

    return gather_kernel(table, idx)


def _sample_shard(coords, features):
    n = coords.shape[0]
    cx = coords[:, :, 0]
    cy = coords[:, :, 1]
    cz = coords[:, :, 2]
    gidx, ox, oy, oz = _fps(cx, cy, cz)
    coords_out = jnp.stack([ox, oy, oz], axis=-1)
    depth = features.shape[-1]
    feats_flat = features.reshape(n * _L, depth)
    feats_pad = jnp.pad(feats_flat, ((0, 0), (0, 128 - depth)))
    feats_out = _sc_gather(feats_pad, gidx.reshape(1, n * _M))
    feats_out = feats_out[:, :depth].reshape(n, _M, depth)
    return coords_out, feats_out


def kernel(coords, features):
    return _sample_shard(coords, features)


# speedup vs baseline: 2.6814x; 1.6761x over previous
import functools

import jax
import jax.numpy as jnp
from jax import lax
from jax.experimental import pallas as pl
from jax.experimental.pallas import tpu as pltpu
from jax.experimental.pallas import tpu_sc as plsc
from jax.sharding import PartitionSpec as P

_L = 4096
_M = 1024
_W = 128


_H = 8
_T = _L // 128
_G = 8


def _fps_kernel(n, cx_ref, cy_ref, cz_ref,
                gidx_ref, ox_ref, oy_ref, oz_ref,
                dist_ref, rm_ref, rt_ref, rx_ref, ry_ref, rz_ref):
    j = pl.program_id(0)
    halves = [(h, min(_H, n - h)) for h in range(0, n, _H)]
    is0 = j == 0

    def sweep(rows, hn, lx, ly, lz):
        groups = []
        for g in range(0, _T, _G):
            acc = None
            for t in range(g, g + _G):
                sl = slice(t * 128, (t + 1) * 128)
                cxt = cx_ref[rows, sl]
                cyt = cy_ref[rows, sl]
                czt = cz_ref[rows, sl]
                dxt = cxt - lx
                dyt = cyt - ly
                dzt = czt - lz
                dt = dxt * dxt + dyt * dyt + dzt * dzt
                dmt = jnp.minimum(dist_ref[rows, sl], dt)
                dist_ref[rows, sl] = dmt
                if acc is None:
                    acc = (dmt, jnp.full((hn, 128), t, jnp.int32),
                           cxt, cyt, czt)
                else:
                    gt = dmt > acc[0]
                    acc = (jnp.where(gt, dmt, acc[0]),
                           jnp.where(gt, t, acc[1]),
                           jnp.where(gt, cxt, acc[2]),
                           jnp.where(gt, cyt, acc[3]),
                           jnp.where(gt, czt, acc[4]))
            groups.append(acc)
        while len(groups) > 1:
            nxt_groups = []
            for a, b in zip(groups[::2], groups[1::2]):
                gt = b[0] > a[0]
                nxt_groups.append(tuple(jnp.where(gt, bb, aa)
                                        for aa, bb in zip(a, b)))
            groups = nxt_groups
        return groups[0]

    def extract(hn, runs, lane):
        rm, rt, rx, ry, rz = runs
        l = jnp.argmax(rm, axis=1, keepdims=True).astype(jnp.int32)
        ohm = lane == l
        ts = jnp.sum(jnp.where(ohm, rt, 0), axis=1, keepdims=True)
        nlx = jnp.sum(jnp.where(ohm, rx, 0.0), axis=1, keepdims=True)
        nly = jnp.sum(jnp.where(ohm, ry, 0.0), axis=1, keepdims=True)
        nlz = jnp.sum(jnp.where(ohm, rz, 0.0), axis=1, keepdims=True)
        jstar = l * _T + ts
        return jstar, nlx, nly, nlz

    @pl.when(is0)
    def _init():
        dist_ref[...] = jnp.full((n, _L), jnp.inf, jnp.float32)
        for h, hn in halves:
            rows = slice(h, h + hn)
            runs0 = sweep(rows, hn,
                          cx_ref[rows, 0:1], cy_ref[rows, 0:1],
                          cz_ref[rows, 0:1])
            rm_ref[rows, :] = runs0[0]
            rt_ref[rows, :] = runs0[1]
            rx_ref[rows, :] = runs0[2]
            ry_ref[rows, :] = runs0[3]
            rz_ref[rows, :] = runs0[4]

    def half_state0(h, hn):
        rows = slice(h, h + hn)
        lane = lax.broadcasted_iota(jnp.int32, (hn, _W), 1)
        base = (lax.broadcasted_iota(jnp.int32, (hn, 1), 0) + h) * _L
        seed = is0 & (lane == 0)
        gbuf0 = jnp.where(seed, base, 0)
        xbuf0 = jnp.where(seed, cx_ref[rows, 0:1], 0.0)
        ybuf0 = jnp.where(seed, cy_ref[rows, 0:1], 0.0)
        zbuf0 = jnp.where(seed, cz_ref[rows, 0:1], 0.0)
        runs = (rm_ref[rows, :], rt_ref[rows, :], rx_ref[rows, :],
                ry_ref[rows, :], rz_ref[rows, :])
        return (gbuf0, xbuf0, ybuf0, zbuf0) + runs

    def half_step(k, h, hn, st):
        gbuf, xbuf, ybuf, zbuf = st[:4]
        runs = st[4:]
        rows = slice(h, h + hn)
        lane = lax.broadcasted_iota(jnp.int32, (hn, _W), 1)
        base = (lax.broadcasted_iota(jnp.int32, (hn, 1), 0) + h) * _L
        jstar, nlx, nly, nlz = extract(hn, runs, lane)
        m = lane == k
        gbuf = jnp.where(m, jstar + base, gbuf)
        xbuf = jnp.where(m, nlx, xbuf)
        ybuf = jnp.where(m, nly, ybuf)
        zbuf = jnp.where(m, nlz, zbuf)
        runs = sweep(rows, hn, nlx, nly, nlz)
        return (gbuf, xbuf, ybuf, zbuf) + runs

    def body(k, carry):
        return tuple(half_step(k, h, hn, st)
                     for (h, hn), st in zip(halves, carry))

    start = jnp.where(is0, 1, 0)
    final = lax.fori_loop(
        start, _W, body,
        tuple(half_state0(h, hn) for h, hn in halves))

    for (h, hn), st in zip(halves, final):
        rows = slice(h, h + hn)
        gidx_ref[rows, :] = st[0]
        ox_ref[rows, :] = st[1]
        oy_ref[rows, :] = st[2]
        oz_ref[rows, :] = st[3]
        rm_ref[rows, :] = st[4]
        rt_ref[rows, :] = st[5]
        rx_ref[rows, :] = st[6]
        ry_ref[rows, :] = st[7]
        rz_ref[rows, :] = st[8]


def _fps(cx, cy, cz):
    n = cx.shape[0]
    out_block = pl.BlockSpec((n, _W), lambda j: (0, j))
    return pl.pallas_call(
        functools.partial(_fps_kernel, n),
        grid=(_M // _W,),
        in_specs=[pl.BlockSpec((n, _L), lambda j: (0, 0))] * 3,
        out_specs=[out_block] * 4,
        out_shape=[
            jax.ShapeDtypeStruct((n, _M), jnp.int32),
            jax.ShapeDtypeStruct((n, _M), jnp.float32),
            jax.ShapeDtypeStruct((n, _M), jnp.float32),
            jax.ShapeDtypeStruct((n, _M), jnp.float32),
        ],
        scratch_shapes=[pltpu.VMEM((n, _L), jnp.float32),
                        pltpu.VMEM((n, 128), jnp.float32),
                        pltpu.VMEM((n, 128), jnp.int32),
                        pltpu.VMEM((n, 128), jnp.float32),
                        pltpu.VMEM((n, 128), jnp.float32),
                        pltpu.VMEM((n, 128), jnp.float32)],
    )(cx, cy, cz)


def _sc_gather(table, idx):
    num_idx = idx.shape[1]
    depth = table.shape[1]
    window = 128
    mesh = plsc.VectorSubcoreMesh(core_axis_name="core",
                                  subcore_axis_name="subcore")

    @pl.kernel(out_type=jax.ShapeDtypeStruct((num_idx, depth), table.dtype),
               mesh=mesh)
    def gather_kernel(x_hbm, i_hbm, o_hbm):
        def body(i_vmem, o_vmem):
            pltpu.sync_copy(x_hbm.at[i_vmem.at[0]], o_vmem)

        pltpu.emit_pipeline(
            body,
            grid=(num_idx // window,),
            in_specs=[pl.BlockSpec((1, window), index_map=lambda i: (0, i))],
            out_specs=[pl.BlockSpec((window, depth),
                                    index_map=lambda i: (i, 0))],
            core_axis_name=("core", "subcore"),
            dimension_semantics=(pltpu.PARALLEL,),
        )(i_hbm, o_hbm)

    return gather_kernel(table, idx)


def _sample_shard(coords, features):
    n = coords.shape[0]
    cperm = coords.reshape(n, 128, _T, 3).transpose(0, 2, 1, 3)
    cperm = cperm.reshape(n, _L, 3)
    cx = cperm[:, :, 0]
    cy = cperm[:, :, 1]
    cz = cperm[:, :, 2]
    gidx, ox, oy, oz = _fps(cx, cy, cz)
    coords_out = jnp.stack([ox, oy, oz], axis=-1)
    depth = features.shape[-1]
    feats_flat = features.reshape(n * _L, depth)
    feats_pad = jnp.pad(feats_flat, ((0, 0), (0, 128 - depth)))
    feats_out = _sc_gather(feats_pad, gidx.reshape(1, n * _M))
    feats_out = feats_out[:, :depth].reshape(n, _M, depth)
    return coords_out, feats_out


def kernel(coords, features):
    return _sample_shard(coords, features)


# speedup vs baseline: 3.2225x; 1.2018x over previous
import functools

import jax
import jax.numpy as jnp
from jax import lax
from jax.experimental import pallas as pl
from jax.experimental.pallas import tpu as pltpu
from jax.experimental.pallas import tpu_sc as plsc
from jax.sharding import PartitionSpec as P

_L = 4096
_M = 1024
_W = 128


_H = 8
_T = _L // 128
_G = 8


def _fps_kernel(n, cx_ref, cy_ref, cz_ref,
                gidx_ref, ox_ref, oy_ref, oz_ref,
                dist_ref, rm_ref, rt_ref, rx_ref, ry_ref, rz_ref,
                bx_ref, by_ref, bz_ref):
    j = pl.program_id(0)
    nh = n // 2
    rows_a = slice(0, nh)
    rows_b = slice(nh, n)
    is0 = j == 0

    def sweep(rows, hn, lx, ly, lz):
        groups = []
        for g in range(0, _T, _G):
            acc = None
            for t in range(g, g + _G):
                sl = slice(t * 128, (t + 1) * 128)
                cxt = cx_ref[rows, sl]
                cyt = cy_ref[rows, sl]
                czt = cz_ref[rows, sl]
                dxt = cxt - lx
                dyt = cyt - ly
                dzt = czt - lz
                dt = dxt * dxt + dyt * dyt + dzt * dzt
                dmt = jnp.minimum(dist_ref[rows, sl], dt)
                dist_ref[rows, sl] = dmt
                if acc is None:
                    acc = (dmt, jnp.full((hn, 128), t, jnp.int32),
                           cxt, cyt, czt)
                else:
                    gt = dmt > acc[0]
                    acc = (jnp.where(gt, dmt, acc[0]),
                           jnp.where(gt, t, acc[1]),
                           jnp.where(gt, cxt, acc[2]),
                           jnp.where(gt, cyt, acc[3]),
                           jnp.where(gt, czt, acc[4]))
            groups.append(acc)
        while len(groups) > 1:
            nxt_groups = []
            for a, b in zip(groups[::2], groups[1::2]):
                gt = b[0] > a[0]
                nxt_groups.append(tuple(jnp.where(gt, bb, aa)
                                        for aa, bb in zip(a, b)))
            groups = nxt_groups
        return groups[0]

    def extract(hn, runs, lane):
        rm, rt, rx, ry, rz = runs
        l = jnp.argmax(rm, axis=1, keepdims=True).astype(jnp.int32)
        ohm = lane == l
        ts = jnp.sum(jnp.where(ohm, rt, 0), axis=1, keepdims=True)
        nlx = jnp.sum(jnp.where(ohm, rx, 0.0), axis=1, keepdims=True)
        nly = jnp.sum(jnp.where(ohm, ry, 0.0), axis=1, keepdims=True)
        nlz = jnp.sum(jnp.where(ohm, rz, 0.0), axis=1, keepdims=True)
        jstar = l * _T + ts
        return jstar, nlx, nly, nlz

    @pl.when(is0)
    def _init():
        dist_ref[...] = jnp.full((n, _L), jnp.inf, jnp.float32)
        runs0 = sweep(rows_a, nh,
                      cx_ref[rows_a, 0:1], cy_ref[rows_a, 0:1],
                      cz_ref[rows_a, 0:1])
        rm_ref[rows_a, :] = runs0[0]
        rt_ref[rows_a, :] = runs0[1]
        rx_ref[rows_a, :] = runs0[2]
        ry_ref[rows_a, :] = runs0[3]
        rz_ref[rows_a, :] = runs0[4]
        bx_ref[rows_b, :] = jnp.broadcast_to(cx_ref[rows_b, 0:1], (nh, 128))
        by_ref[rows_b, :] = jnp.broadcast_to(cy_ref[rows_b, 0:1], (nh, 128))
        bz_ref[rows_b, :] = jnp.broadcast_to(cz_ref[rows_b, 0:1], (nh, 128))

    lane = lax.broadcasted_iota(jnp.int32, (nh, _W), 1)
    base_a = lax.broadcasted_iota(jnp.int32, (nh, 1), 0) * _L
    base_b = (lax.broadcasted_iota(jnp.int32, (nh, 1), 0) + nh) * _L
    seed = is0 & (lane == 0)

    carry_a = (jnp.where(seed, base_a, 0),
               jnp.where(seed, cx_ref[rows_a, 0:1], 0.0),
               jnp.where(seed, cy_ref[rows_a, 0:1], 0.0),
               jnp.where(seed, cz_ref[rows_a, 0:1], 0.0),
               rm_ref[rows_a, :], rt_ref[rows_a, :], rx_ref[rows_a, :],
               ry_ref[rows_a, :], rz_ref[rows_a, :])
    carry_b = (jnp.where(seed, base_b, 0),
               jnp.where(seed, cx_ref[rows_b, 0:1], 0.0),
               jnp.where(seed, cy_ref[rows_b, 0:1], 0.0),
               jnp.where(seed, cz_ref[rows_b, 0:1], 0.0),
               bx_ref[rows_b, :], by_ref[rows_b, :], bz_ref[rows_b, :])

    def body(k, carry):
        ca, cb = carry
        ga, xa, ya, za = ca[:4]
        runs_a = ca[4:]
        ja, ax, ay, az = extract(nh, runs_a, lane)
        m = lane == k
        ga = jnp.where(m, ja + base_a, ga)
        xa = jnp.where(m, ax, xa)
        ya = jnp.where(m, ay, ya)
        za = jnp.where(m, az, za)
        runs_a = sweep(rows_a, nh, ax, ay, az)
        gb, xb, yb, zb = cb[:4]
        blx, bly, blz = cb[4:]
        runs_b = sweep(rows_b, nh, blx, bly, blz)
        jb, bx, by, bz = extract(nh, runs_b, lane)
        gb = jnp.where(m, jb + base_b, gb)
        xb = jnp.where(m, bx, xb)
        yb = jnp.where(m, by, yb)
        zb = jnp.where(m, bz, zb)
        cb_new = (gb, xb, yb, zb,
                  jnp.broadcast_to(bx, (nh, 128)),
                  jnp.broadcast_to(by, (nh, 128)),
                  jnp.broadcast_to(bz, (nh, 128)))
        return ((ga, xa, ya, za) + runs_a, cb_new)

    start = jnp.where(is0, 1, 0)
    fa, fb = lax.fori_loop(start, _W, body, (carry_a, carry_b))

    gidx_ref[rows_a, :] = fa[0]
    ox_ref[rows_a, :] = fa[1]
    oy_ref[rows_a, :] = fa[2]
    oz_ref[rows_a, :] = fa[3]
    rm_ref[rows_a, :] = fa[4]
    rt_ref[rows_a, :] = fa[5]
    rx_ref[rows_a, :] = fa[6]
    ry_ref[rows_a, :] = fa[7]
    rz_ref[rows_a, :] = fa[8]
    gidx_ref[rows_b, :] = fb[0]
    ox_ref[rows_b, :] = fb[1]
    oy_ref[rows_b, :] = fb[2]
    oz_ref[rows_b, :] = fb[3]
    bx_ref[rows_b, :] = fb[4]
    by_ref[rows_b, :] = fb[5]
    bz_ref[rows_b, :] = fb[6]


def _fps(cx, cy, cz):
    n = cx.shape[0]
    out_block = pl.BlockSpec((n, _W), lambda j: (0, j))
    return pl.pallas_call(
        functools.partial(_fps_kernel, n),
        grid=(_M // _W,),
        in_specs=[pl.BlockSpec((n, _L), lambda j: (0, 0))] * 3,
        out_specs=[out_block] * 4,
        out_shape=[
            jax.ShapeDtypeStruct((n, _M), jnp.int32),
            jax.ShapeDtypeStruct((n, _M), jnp.float32),
            jax.ShapeDtypeStruct((n, _M), jnp.float32),
            jax.ShapeDtypeStruct((n, _M), jnp.float32),
        ],
        scratch_shapes=[pltpu.VMEM((n, _L), jnp.float32),
                        pltpu.VMEM((n, 128), jnp.float32),
                        pltpu.VMEM((n, 128), jnp.int32),
                        pltpu.VMEM((n, 128), jnp.float32),
                        pltpu.VMEM((n, 128), jnp.float32),
                        pltpu.VMEM((n, 128), jnp.float32),
                        pltpu.VMEM((n, 128), jnp.float32),
                        pltpu.VMEM((n, 128), jnp.float32),
                        pltpu.VMEM((n, 128), jnp.float32)],
    )(cx, cy, cz)


def _sc_gather(table, idx):
    num_idx = idx.shape[1]
    depth = table.shape[1]
    window = 128
    mesh = plsc.VectorSubcoreMesh(core_axis_name="core",
                                  subcore_axis_name="subcore")

    @pl.kernel(out_type=jax.ShapeDtypeStruct((num_idx, depth), table.dtype),
               mesh=mesh)
    def gather_kernel(x_hbm, i_hbm, o_hbm):
        def body(i_vmem, o_vmem):
            pltpu.sync_copy(x_hbm.at[i_vmem.at[0]], o_vmem)

        pltpu.emit_pipeline(
            body,
            grid=(num_idx // window,),
            in_specs=[pl.BlockSpec((1, window), index_map=lambda i: (0, i))],
            out_specs=[pl.BlockSpec((window, depth),
                                    index_map=lambda i: (i, 0))],
            core_axis_name=("core", "subcore"),
            dimension_semantics=(pltpu.PARALLEL,),
        )(i_hbm, o_hbm)

    return gather_kernel(table, idx)


def _sample_shard(coords, features):
    n = coords.shape[0]
    cperm = coords.reshape(n, 128, _T, 3).transpose(0, 2, 1, 3)
    cperm = cperm.reshape(n, _L, 3)
    cx = cperm[:, :, 0]
    cy = cperm[:, :, 1]
    cz = cperm[:, :, 2]
    gidx, ox, oy, oz = _fps(cx, cy, cz)
    coords_out = jnp.stack([ox, oy, oz], axis=-1)
    depth = features.shape[-1]
    feats_flat = features.reshape(n * _L, depth)
    feats_pad = jnp.pad(feats_flat, ((0, 0), (0, 128 - depth)))
    feats_out = _sc_gather(feats_pad, gidx.reshape(1, n * _M))
    feats_out = feats_out[:, :depth].reshape(n, _M, depth)
    return coords_out, feats_out


def kernel(coords, features):
    return _sample_shard(coords, features)
